# baseline (speedup 1.0000x reference)
import functools

import jax
import jax.numpy as jnp
from jax import lax
from jax.experimental import pallas as pl
from jax.experimental.pallas import tpu as pltpu
from jax.experimental.pallas import tpu_sc as plsc

PS = 2.0
XMIN = -60.0
YMIN = -60.0
GW = 60
GH = 60
NPIL = GW * GH
EPS = 1e-5

B = 4
N = 131072
PT = 2048
NT = N // PT
NTOT = float(B * N)
G64 = 64
CH = 128


def _pillar_onehots(pts):
    x = pts[0:1, :]
    y = pts[1:2, :]
    pxi = ((x - XMIN) / PS).astype(jnp.int32)
    pyi = ((y - YMIN) / PS).astype(jnp.int32)
    valid = ((pxi >= 0) & (pxi < GW) & (pyi >= 0) & (pyi < GH))
    pxi = jnp.clip(pxi, 0, GW - 1)
    pyi = jnp.clip(pyi, 0, GH - 1)
    giota = lax.broadcasted_iota(jnp.int32, (G64, 1), 0)
    oxT = (giota == pxi).astype(jnp.float32)
    oyT = (giota == pyi).astype(jnp.float32)
    return oxT, oyT, valid.astype(jnp.float32), pxi, pyi


def _stage_a_body(pts_ref, psum_ref, pidx_ref):
    t = pl.program_id(1)
    pts = pts_ref[0]
    oxT, oyT, valid, pxi, pyi = _pillar_onehots(pts)
    pidx_ref[0, 0] = (pyi * GW + pxi).astype(jnp.int32)

    oxv = oxT * valid
    dn = (((1,), (1,)), ((), ()))

    @pl.when(t == 0)
    def _():
        psum_ref[...] = jnp.zeros_like(psum_ref)

    for c in range(4):
        v = pts[c:c + 1, :] if c < 3 else jnp.ones_like(valid)
        contrib = lax.dot_general(oyT, oxv * v, dn,
                                  preferred_element_type=jnp.float32)
        psum_ref[0, c] += contrib


def _stage_b_body(pts_ref, psum_ref, aug_ref, m_ref):
    bi = pl.program_id(0)
    t = pl.program_id(1)
    pts = pts_ref[0]
    oxT, oyT, valid, pxi, pyi = _pillar_onehots(pts)

    cnt = psum_ref[0, 3]
    inv = 1.0 / jnp.maximum(cnt, 1.0)
    dn0 = (((0,), (0,)), ((), ()))
    gm = []
    for c in range(3):
        pm = psum_ref[0, c] * inv
        u = lax.dot_general(pm, oyT, dn0,
                            preferred_element_type=jnp.float32)
        gm.append(jnp.sum(oxT * u, axis=0, keepdims=True))

    cx = XMIN + (pxi.astype(jnp.float32) + 0.5) * PS
    cy = YMIN + (pyi.astype(jnp.float32) + 0.5) * PS
    om = pts[0:3, :] - jnp.concatenate(gm, axis=0)
    oc = pts[0:2, :] - jnp.concatenate([cx, cy], axis=0)
    z2 = jnp.zeros((2, pts.shape[1]), jnp.float32)
    augh = jnp.concatenate([pts, om, oc, valid, z2], axis=0) * valid
    aug_ref[0] = augh

    dn1 = (((1,), (1,)), ((), ()))
    m = lax.dot_general(augh, augh, dn1, preferred_element_type=jnp.float32)

    @pl.when((bi == 0) & (t == 0))
    def _():
        m_ref[...] = jnp.zeros_like(m_ref)

    m_ref[...] += m


def _stage_c_body(aug_ref, m_ref, w_ref, p_ref, feat_ref):
    mm = m_ref[...] * (1.0 / NTOT)
    w = w_ref[...]
    bb = p_ref[0:1, :]
    gg = p_ref[1:2, :]
    be = p_ref[2:3, :]

    cw = lax.dot_general(mm, w, (((1,), (0,)), ((), ())),
                         preferred_element_type=jnp.float32)
    ef2 = jnp.sum(w * cw, axis=0, keepdims=True)
    colmask = (lax.broadcasted_iota(jnp.int32, (1, 16), 1) < 13)
    s1 = mm[13:14, :] * colmask.astype(jnp.float32)
    mu0 = lax.dot_general(s1, w, (((1,), (0,)), ((), ())),
                          preferred_element_type=jnp.float32)
    mean = mu0 + bb
    var = ef2 + 2.0 * bb * mu0 + bb * bb - mean * mean
    scale = gg * lax.rsqrt(var + EPS)
    shift = be - mean * scale
    rowmask = (lax.broadcasted_iota(jnp.int32, (16, 1), 0) == 13)
    weff = w * scale + rowmask.astype(jnp.float32) * shift

    augh = aug_ref[0]
    feat = lax.dot_general(augh, weff, (((0,), (0,)), ((), ())),
                           preferred_element_type=jnp.float32)
    feat_ref[0] = jnp.maximum(feat, 0.0)


PR = NPIL // 2
TROWS = PR + 8
TW = TROWS * 64


def _scatter_max(pidx1d, feat1d):
    mesh = plsc.VectorSubcoreMesh(core_axis_name="c", subcore_axis_name="s")
    Q = 4
    npts = N // Q

    @functools.partial(
        pl.kernel,
        out_type=jax.ShapeDtypeStruct((32 * TW,), jnp.float32),
        mesh=mesh,
        scratch_types=[
            pltpu.VMEM((TW,), jnp.float32),
            pltpu.VMEM((CH,), jnp.int32),
            pltpu.VMEM((CH * 64,), jnp.float32),
        ],
    )
    def sc_kernel(idx_hbm, feat_hbm, out_hbm, tbl, idxb, rowb):
        ci = lax.axis_index("c")
        si = lax.axis_index("s")
        wid = si * 2 + ci
        q = wid // 8
        r = wid % 8
        bi = r // 2
        pr = r % 2
        pbase = pr * PR

        def zbody(i, carry):
            tbl[pl.ds(i * 16, 16)] = jnp.zeros((16,), jnp.float32)
            return carry

        lax.fori_loop(0, TW // 16, zbody, 0)

        def chunk(g, carry):
            pstart = bi * N + q * npts + g * CH
            pltpu.sync_copy(idx_hbm.at[pl.ds(pstart, CH)], idxb)
            pltpu.sync_copy(feat_hbm.at[pl.ds(pstart * 64, CH * 64)], rowb)

            def grp(k, c2):
                iv = idxb[pl.ds(k * 16, 16)]
                pis = [iv[jj] for jj in range(16)]
                p0 = pis[0]
                same = pis[0] == pis[1]
                for jj in range(2, 16):
                    same = same & (p0 == pis[jj])
                rb = k * 16 * 64

                def fast(_):
                    local = p0 - pbase
                    ok = (local >= 0) & (local < PR)
                    ta = jnp.where(ok, local, PR) * 64
                    for c in range(4):
                        m = rowb[pl.ds(rb + c * 16, 16)]
                        for jj in range(1, 16):
                            m = jnp.maximum(
                                m, rowb[pl.ds(rb + jj * 64 + c * 16, 16)])
                        cur = tbl[pl.ds(ta + c * 16, 16)]
                        tbl[pl.ds(ta + c * 16, 16)] = jnp.maximum(cur, m)
                    return 0

                def slow(_):
                    for jj in range(16):
                        local = pis[jj] - pbase
                        ok = (local >= 0) & (local < PR)
                        ta = jnp.where(ok, local, PR) * 64
                        ra = rb + jj * 64
                        for c in range(4):
                            row = rowb[pl.ds(ra + c * 16, 16)]
                            cur = tbl[pl.ds(ta + c * 16, 16)]
                            tbl[pl.ds(ta + c * 16, 16)] = \
                                jnp.maximum(cur, row)
                    return 0

                lax.cond(same, fast, slow, 0)
                return c2

            lax.fori_loop(0, CH // 16, grp, 0)
            return carry

        lax.fori_loop(0, npts // CH, chunk, 0)
        pltpu.sync_copy(tbl, out_hbm.at[pl.ds(wid * TW, TW)])

    return sc_kernel(pidx1d, feat1d)


def _conv_bn_relu(xp, wt_ref, pk, stride, hout, cout):
    cin = xp.shape[-1]
    h = xp.shape[1] - 2
    hr = 2 * hout if stride == 2 else h
    rows = B * hout * h if stride == 2 else B * h * h
    acc = jnp.zeros((rows, cout), jnp.float32)
    for di in range(3):
        for dj in range(3):
            sl = lax.slice(xp, (0, di, dj, 0), (B, di + hr, dj + h, cin))
            if stride == 2:
                sl = sl.reshape(B, hout, 2, h, cin)[:, :, 0, :, :]
            acc += lax.dot_general(
                sl.reshape(rows, cin), wt_ref[di * 3 + dj],
                (((1,), (0,)), ((), ())), preferred_element_type=jnp.float32)
    if stride == 2:
        o = lax.broadcasted_iota(jnp.int32, (hout * hout, 1), 0)
        i = lax.broadcasted_iota(jnp.int32, (1, hout * h), 1)
        target = (o // hout) * h + 2 * (o % hout)
        sel = (i == target).astype(jnp.float32)
        y = acc.reshape(B, hout * h, cout)
        zs = [lax.dot_general(sel, y[b], (((1,), (0,)), ((), ())),
                              preferred_element_type=jnp.float32)
              .reshape(1, hout * hout, cout) for b in range(B)]
        acc = jnp.concatenate(zs, axis=0).reshape(B * hout * hout, cout)
    acc += pk[0:1, :]
    mean = jnp.mean(acc, axis=0, keepdims=True)
    var = jnp.mean((acc - mean) * (acc - mean), axis=0, keepdims=True)
    h2 = (acc - mean) * lax.rsqrt(var + EPS) * pk[1:2, :] + pk[2:3, :]
    h2 = jnp.maximum(h2, 0.0)
    return h2.reshape(B, hout, hout, cout)


def _stage_merge_body(pf_ref, out_ref):
    pf4 = pf_ref[...]
    pf = jnp.max(pf4, axis=0)
    out_ref[...] = pf.reshape(B, GH, GW, 64)


def _conv_taps_body(x_ref, w_ref, out_ref, pad, *, stride, hout):
    h = x_ref.shape[1]
    cin = x_ref.shape[3]
    cout = w_ref.shape[2]
    pad[...] = jnp.zeros_like(pad)
    pad[:, 1:h + 1, 1:h + 1, :] = x_ref[...]
    xp = pad[...]
    hr = 2 * hout if stride == 2 else h
    rows = hout * h if stride == 2 else h * h
    acc = jnp.zeros((rows, cout), jnp.float32)
    for di in range(3):
        for dj in range(3):
            sl = lax.slice(xp, (0, di, dj, 0), (1, di + hr, dj + h, cin))
            if stride == 2:
                sl = sl.reshape(1, hout, 2, h, cin)[:, :, 0, :, :]
            acc += lax.dot_general(
                sl.reshape(rows, cin), w_ref[di * 3 + dj],
                (((1,), (0,)), ((), ())), preferred_element_type=jnp.float32)
    if stride == 2:
        o = lax.broadcasted_iota(jnp.int32, (hout * hout, 1), 0)
        i = lax.broadcasted_iota(jnp.int32, (1, hout * h), 1)
        target = (o // hout) * h + 2 * (o % hout)
        sel = (i == target).astype(jnp.float32)
        acc = lax.dot_general(sel, acc, (((1,), (0,)), ((), ())),
                              preferred_element_type=jnp.float32)
    out_ref[0] = acc


def _bn_relu_body(a_ref, pk_ref, out_ref):
    r = a_ref.shape[1]
    c = a_ref.shape[2]
    pk = pk_ref[...]
    a = a_ref[...].reshape(B * r, c) + pk[0:1, :]
    mean = jnp.mean(a, axis=0, keepdims=True)
    var = jnp.mean((a - mean) * (a - mean), axis=0, keepdims=True)
    h = (a - mean) * lax.rsqrt(var + EPS) * pk[1:2, :] + pk[2:3, :]
    out_ref[...] = jnp.maximum(h, 0.0).reshape(B, r, c)


def _conv_grid(x, wt, stride, hout, cout):
    h = x.shape[1]
    cin = x.shape[3]
    rows = hout * hout
    return pl.pallas_call(
        functools.partial(_conv_taps_body, stride=stride, hout=hout),
        grid=(B,),
        in_specs=[
            pl.BlockSpec((1, h, h, cin), lambda b: (b, 0, 0, 0)),
            pl.BlockSpec((9, cin, cout), lambda b: (0, 0, 0)),
        ],
        out_specs=pl.BlockSpec((1, rows, cout), lambda b: (b, 0, 0)),
        out_shape=jax.ShapeDtypeStruct((B, rows, cout), jnp.float32),
        scratch_shapes=[pltpu.VMEM((1, h + 2, h + 2, cin), jnp.float32)],
    )(x, wt)


def _bn_relu(a, pk):
    return pl.pallas_call(
        _bn_relu_body,
        out_shape=jax.ShapeDtypeStruct(a.shape, jnp.float32),
    )(a, pk)


def _stage_e2_body(h2_ref, w3_ref, w4_ref, p3_ref, p4_ref, mlp_ref, p5_ref,
                   out_ref, s2, s3):
    h2 = h2_ref[...]
    s2[...] = jnp.zeros_like(s2)
    s2[:, 1:31, 1:31, :] = h2
    h3 = _conv_bn_relu(s2[...], w3_ref, p3_ref[...], 2, 15, 256)

    s3[...] = jnp.zeros_like(s3)
    s3[:, 1:16, 1:16, :] = h3
    h4 = _conv_bn_relu(s3[...], w4_ref, p4_ref[...], 2, 8, 256)

    gf = jnp.mean(h4.reshape(B, 64, 256), axis=1)
    lat = lax.dot_general(gf, mlp_ref[...], (((1,), (0,)), ((), ())),
                          preferred_element_type=jnp.float32)
    lat = lat + p5_ref[0:1, :]
    mu = jnp.mean(lat, axis=1, keepdims=True)
    var = jnp.mean((lat - mu) * (lat - mu), axis=1, keepdims=True)
    out_ref[...] = (lat - mu) * lax.rsqrt(var + EPS) * p5_ref[1:2, :] \
        + p5_ref[2:3, :]


def kernel(points, pfn_w, pfn_b, pfn_bn_g, pfn_bn_b,
           cw1, cb1, bg1, bb1, cw2, cb2, bg2, bb2,
           cw3, cb3, bg3, bb3, cw4, cb4, bg4, bb4,
           mlp_w, mlp_b, ln_g, ln_b):
    pts_t = jnp.transpose(points, (0, 2, 1))

    psum, pidx4 = pl.pallas_call(
        _stage_a_body,
        grid=(B, NT),
        in_specs=[pl.BlockSpec((1, 8, PT), lambda b, t: (b, 0, t))],
        out_specs=[
            pl.BlockSpec((1, 4, G64, G64), lambda b, t: (b, 0, 0, 0)),
            pl.BlockSpec((1, 1, 1, PT), lambda b, t: (b, t, 0, 0)),
        ],
        out_shape=[
            jax.ShapeDtypeStruct((B, 4, G64, G64), jnp.float32),
            jax.ShapeDtypeStruct((B, NT, 1, PT), jnp.int32),
        ],
    )(pts_t)

    aug_t, mmat = pl.pallas_call(
        _stage_b_body,
        grid=(B, NT),
        in_specs=[
            pl.BlockSpec((1, 8, PT), lambda b, t: (b, 0, t)),
            pl.BlockSpec((1, 4, G64, G64), lambda b, t: (b, 0, 0, 0)),
        ],
        out_specs=[
            pl.BlockSpec((1, 16, PT), lambda b, t: (b, 0, t)),
            pl.BlockSpec((16, 16), lambda b, t: (0, 0)),
        ],
        out_shape=[
            jax.ShapeDtypeStruct((B, 16, N), jnp.float32),
            jax.ShapeDtypeStruct((16, 16), jnp.float32),
        ],
    )(pts_t, psum)

    wt16 = jnp.zeros((16, 64), jnp.float32).at[0:13, :].set(pfn_w.T)
    pfnp = jnp.zeros((8, 64), jnp.float32)
    pfnp = pfnp.at[0].set(pfn_b).at[1].set(pfn_bn_g).at[2].set(pfn_bn_b)

    feat = pl.pallas_call(
        _stage_c_body,
        grid=(B, NT),
        in_specs=[
            pl.BlockSpec((1, 16, PT), lambda b, t: (b, 0, t)),
            pl.BlockSpec((16, 16), lambda b, t: (0, 0)),
            pl.BlockSpec((16, 64), lambda b, t: (0, 0)),
            pl.BlockSpec((8, 64), lambda b, t: (0, 0)),
        ],
        out_specs=pl.BlockSpec((1, PT, 64), lambda b, t: (b, t, 0)),
        out_shape=jax.ShapeDtypeStruct((B, N, 64), jnp.float32),
    )(aug_t, mmat, wt16, pfnp)

    pfeat1d = _scatter_max(pidx4.reshape(-1), feat.reshape(-1))
    pfeat = pfeat1d.reshape(4, B, 2, TROWS, 64)[:, :, :, :PR, :]
    pfeat = pfeat.reshape(4, B, NPIL, 64)

    w1t = jnp.transpose(cw1, (2, 3, 1, 0)).reshape(9, 64, 64)
    w2t = jnp.transpose(cw2, (2, 3, 1, 0)).reshape(9, 64, 128)
    w3t = jnp.transpose(cw3, (2, 3, 1, 0)).reshape(9, 128, 256)
    w4t = jnp.transpose(cw4, (2, 3, 1, 0)).reshape(9, 256, 256)

    def pack3(a, b_, c):
        p = jnp.zeros((8, a.shape[0]), jnp.float32)
        return p.at[0].set(a).at[1].set(b_).at[2].set(c)

    p1 = pack3(cb1, bg1, bb1)
    p2 = pack3(cb2, bg2, bb2)
    p3 = pack3(cb3, bg3, bb3)
    p4 = pack3(cb4, bg4, bb4)
    p5 = pack3(mlp_b, ln_g, ln_b)

    x0 = pl.pallas_call(
        _stage_merge_body,
        out_shape=jax.ShapeDtypeStruct((B, GH, GW, 64), jnp.float32),
    )(pfeat)

    c1 = _conv_grid(x0, w1t, 1, 60, 64)
    h1 = _bn_relu(c1, p1).reshape(B, 60, 60, 64)
    c2 = _conv_grid(h1, w2t, 2, 30, 128)
    h2 = _bn_relu(c2, p2).reshape(B, 30, 30, 128)

    out = pl.pallas_call(
        _stage_e2_body,
        out_shape=jax.ShapeDtypeStruct((B, 256), jnp.float32),
        scratch_shapes=[
            pltpu.VMEM((B, 32, 32, 128), jnp.float32),
            pltpu.VMEM((B, 18, 18, 256), jnp.float32),
        ],
    )(h2, w3t, w4t, p3, p4, jnp.transpose(mlp_w), p5)

    return out


# speedup vs baseline: 1.0878x; 1.0878x over previous
import functools

import jax
import jax.numpy as jnp
from jax import lax
from jax.experimental import pallas as pl
from jax.experimental.pallas import tpu as pltpu
from jax.experimental.pallas import tpu_sc as plsc

PS = 2.0
XMIN = -60.0
YMIN = -60.0
GW = 60
GH = 60
NPIL = GW * GH
EPS = 1e-5

B = 4
N = 131072
PT = 2048
NT = N // PT
NTOT = float(B * N)
G64 = 64
CH = 128


def _pillar_onehots(pts):
    x = pts[0:1, :]
    y = pts[1:2, :]
    pxi = ((x - XMIN) / PS).astype(jnp.int32)
    pyi = ((y - YMIN) / PS).astype(jnp.int32)
    valid = ((pxi >= 0) & (pxi < GW) & (pyi >= 0) & (pyi < GH))
    pxi = jnp.clip(pxi, 0, GW - 1)
    pyi = jnp.clip(pyi, 0, GH - 1)
    giota = lax.broadcasted_iota(jnp.int32, (G64, 1), 0)
    oxT = (giota == pxi).astype(jnp.float32)
    oyT = (giota == pyi).astype(jnp.float32)
    return oxT, oyT, valid.astype(jnp.float32), pxi, pyi


def _stage_a_body(pts_ref, psum_ref, pidx_ref):
    t = pl.program_id(1)
    pts = pts_ref[0]
    oxT, oyT, valid, pxi, pyi = _pillar_onehots(pts)
    pidx_ref[0, 0] = (pyi * GW + pxi).astype(jnp.int32)

    oxv = oxT * valid
    dn = (((1,), (1,)), ((), ()))

    @pl.when(t == 0)
    def _():
        psum_ref[...] = jnp.zeros_like(psum_ref)

    for c in range(4):
        v = pts[c:c + 1, :] if c < 3 else jnp.ones_like(valid)
        contrib = lax.dot_general(oyT, oxv * v, dn,
                                  preferred_element_type=jnp.float32)
        psum_ref[0, c] += contrib


def _stage_b_body(pts_ref, psum_ref, aug_ref, m_ref):
    bi = pl.program_id(0)
    t = pl.program_id(1)
    pts = pts_ref[0]
    oxT, oyT, valid, pxi, pyi = _pillar_onehots(pts)

    cnt = psum_ref[0, 3]
    inv = 1.0 / jnp.maximum(cnt, 1.0)
    dn0 = (((0,), (0,)), ((), ()))
    gm = []
    for c in range(3):
        pm = psum_ref[0, c] * inv
        u = lax.dot_general(pm, oyT, dn0,
                            preferred_element_type=jnp.float32)
        gm.append(jnp.sum(oxT * u, axis=0, keepdims=True))

    cx = XMIN + (pxi.astype(jnp.float32) + 0.5) * PS
    cy = YMIN + (pyi.astype(jnp.float32) + 0.5) * PS
    om = pts[0:3, :] - jnp.concatenate(gm, axis=0)
    oc = pts[0:2, :] - jnp.concatenate([cx, cy], axis=0)
    z2 = jnp.zeros((2, pts.shape[1]), jnp.float32)
    augh = jnp.concatenate([pts, om, oc, valid, z2], axis=0) * valid
    aug_ref[0] = augh

    dn1 = (((1,), (1,)), ((), ()))
    m = lax.dot_general(augh, augh, dn1, preferred_element_type=jnp.float32)

    @pl.when((bi == 0) & (t == 0))
    def _():
        m_ref[...] = jnp.zeros_like(m_ref)

    m_ref[...] += m


def _stage_c_body(aug_ref, m_ref, w_ref, p_ref, feat_ref):
    mm = m_ref[...] * (1.0 / NTOT)
    w = w_ref[...]
    bb = p_ref[0:1, :]
    gg = p_ref[1:2, :]
    be = p_ref[2:3, :]

    cw = lax.dot_general(mm, w, (((1,), (0,)), ((), ())),
                         preferred_element_type=jnp.float32)
    ef2 = jnp.sum(w * cw, axis=0, keepdims=True)
    colmask = (lax.broadcasted_iota(jnp.int32, (1, 16), 1) < 13)
    s1 = mm[13:14, :] * colmask.astype(jnp.float32)
    mu0 = lax.dot_general(s1, w, (((1,), (0,)), ((), ())),
                          preferred_element_type=jnp.float32)
    mean = mu0 + bb
    var = ef2 + 2.0 * bb * mu0 + bb * bb - mean * mean
    scale = gg * lax.rsqrt(var + EPS)
    shift = be - mean * scale
    rowmask = (lax.broadcasted_iota(jnp.int32, (16, 1), 0) == 13)
    weff = w * scale + rowmask.astype(jnp.float32) * shift

    augh = aug_ref[0]
    feat = lax.dot_general(augh, weff, (((0,), (0,)), ((), ())),
                           preferred_element_type=jnp.float32)
    feat_ref[0] = jnp.maximum(feat, 0.0)


PR = NPIL // 2
TROWS = PR + 8
TW = TROWS * 64


def _scatter_max(pidx1d, feat1d):
    mesh = plsc.VectorSubcoreMesh(core_axis_name="c", subcore_axis_name="s")
    Q = 4
    npts = N // Q

    @functools.partial(
        pl.kernel,
        out_type=jax.ShapeDtypeStruct((32 * TW,), jnp.float32),
        mesh=mesh,
        scratch_types=[
            pltpu.VMEM((TW,), jnp.float32),
            pltpu.VMEM((16 * CH,), jnp.int32),
            pltpu.VMEM((CH * 64,), jnp.float32),
        ],
    )
    def sc_kernel(idx_hbm, feat_hbm, out_hbm, tbl, idxb, rowb):
        ci = lax.axis_index("c")
        si = lax.axis_index("s")
        wid = si * 2 + ci
        q = wid // 8
        r = wid % 8
        bi = r // 2
        pr = r % 2
        pbase = pr * PR

        def zbody(i, carry):
            tbl[pl.ds(i * 16, 16)] = jnp.zeros((16,), jnp.float32)
            return carry

        lax.fori_loop(0, TW // 16, zbody, 0)

        def sup(sg, carry):
            sbase = bi * N + q * npts + sg * (16 * CH)
            pltpu.sync_copy(idx_hbm.at[pl.ds(sbase, 16 * CH)], idxb)

            def chunk(g, c3):
                pstart = sbase + g * CH
                pltpu.sync_copy(feat_hbm.at[pl.ds(pstart * 64, CH * 64)],
                                rowb)

                def grp(k, c2):
                    iv = idxb[pl.ds(g * CH + k * 16, 16)]
                    for jj in range(16):
                        local = iv[jj] - pbase
                        ok = (local >= 0) & (local < PR)
                        ta = jnp.where(ok, local, PR) * 64
                        ra = (k * 16 + jj) * 64
                        for c in range(4):
                            row = rowb[pl.ds(ra + c * 16, 16)]
                            cur = tbl[pl.ds(ta + c * 16, 16)]
                            tbl[pl.ds(ta + c * 16, 16)] = \
                                jnp.maximum(cur, row)
                    return c2

                lax.fori_loop(0, CH // 16, grp, 0)
                return c3

            lax.fori_loop(0, 16, chunk, 0)
            return carry

        lax.fori_loop(0, npts // (16 * CH), sup, 0)
        pltpu.sync_copy(tbl, out_hbm.at[pl.ds(wid * TW, TW)])

    return sc_kernel(pidx1d, feat1d)


def _conv_bn_relu(xp, wt_ref, pk, stride, hout, cout):
    cin = xp.shape[-1]
    h = xp.shape[1] - 2
    hr = 2 * hout if stride == 2 else h
    rows = B * hout * h if stride == 2 else B * h * h
    acc = jnp.zeros((rows, cout), jnp.float32)
    for di in range(3):
        for dj in range(3):
            sl = lax.slice(xp, (0, di, dj, 0), (B, di + hr, dj + h, cin))
            if stride == 2:
                sl = sl.reshape(B, hout, 2, h, cin)[:, :, 0, :, :]
            acc += lax.dot_general(
                sl.reshape(rows, cin), wt_ref[di * 3 + dj],
                (((1,), (0,)), ((), ())), preferred_element_type=jnp.float32)
    if stride == 2:
        o = lax.broadcasted_iota(jnp.int32, (hout * hout, 1), 0)
        i = lax.broadcasted_iota(jnp.int32, (1, hout * h), 1)
        target = (o // hout) * h + 2 * (o % hout)
        sel = (i == target).astype(jnp.float32)
        y = acc.reshape(B, hout * h, cout)
        zs = [lax.dot_general(sel, y[b], (((1,), (0,)), ((), ())),
                              preferred_element_type=jnp.float32)
              .reshape(1, hout * hout, cout) for b in range(B)]
        acc = jnp.concatenate(zs, axis=0).reshape(B * hout * hout, cout)
    acc += pk[0:1, :]
    mean = jnp.mean(acc, axis=0, keepdims=True)
    var = jnp.mean((acc - mean) * (acc - mean), axis=0, keepdims=True)
    h2 = (acc - mean) * lax.rsqrt(var + EPS) * pk[1:2, :] + pk[2:3, :]
    h2 = jnp.maximum(h2, 0.0)
    return h2.reshape(B, hout, hout, cout)


def _stage_merge_body(pf_ref, out_ref):
    pf4 = pf_ref[...]
    pf = jnp.max(pf4, axis=0)
    out_ref[...] = pf.reshape(B, GH, GW, 64)


def _conv_taps_body(x_ref, w_ref, out_ref, pad, *, stride, hout):
    h = x_ref.shape[1]
    cin = x_ref.shape[3]
    cout = w_ref.shape[2]
    pad[...] = jnp.zeros_like(pad)
    pad[:, 1:h + 1, 1:h + 1, :] = x_ref[...]
    xp = pad[...]
    hr = 2 * hout if stride == 2 else h
    rows = hout * h if stride == 2 else h * h
    acc = jnp.zeros((rows, cout), jnp.float32)
    for di in range(3):
        for dj in range(3):
            sl = lax.slice(xp, (0, di, dj, 0), (1, di + hr, dj + h, cin))
            if stride == 2:
                sl = sl.reshape(1, hout, 2, h, cin)[:, :, 0, :, :]
            acc += lax.dot_general(
                sl.reshape(rows, cin), w_ref[di * 3 + dj],
                (((1,), (0,)), ((), ())), preferred_element_type=jnp.float32)
    if stride == 2:
        o = lax.broadcasted_iota(jnp.int32, (hout * hout, 1), 0)
        i = lax.broadcasted_iota(jnp.int32, (1, hout * h), 1)
        target = (o // hout) * h + 2 * (o % hout)
        sel = (i == target).astype(jnp.float32)
        acc = lax.dot_general(sel, acc, (((1,), (0,)), ((), ())),
                              preferred_element_type=jnp.float32)
    out_ref[0] = acc


def _bn_relu_body(a_ref, pk_ref, out_ref):
    r = a_ref.shape[1]
    c = a_ref.shape[2]
    pk = pk_ref[...]
    a = a_ref[...].reshape(B * r, c) + pk[0:1, :]
    mean = jnp.mean(a, axis=0, keepdims=True)
    var = jnp.mean((a - mean) * (a - mean), axis=0, keepdims=True)
    h = (a - mean) * lax.rsqrt(var + EPS) * pk[1:2, :] + pk[2:3, :]
    out_ref[...] = jnp.maximum(h, 0.0).reshape(B, r, c)


def _conv_grid(x, wt, stride, hout, cout):
    h = x.shape[1]
    cin = x.shape[3]
    rows = hout * hout
    return pl.pallas_call(
        functools.partial(_conv_taps_body, stride=stride, hout=hout),
        grid=(B,),
        in_specs=[
            pl.BlockSpec((1, h, h, cin), lambda b: (b, 0, 0, 0)),
            pl.BlockSpec((9, cin, cout), lambda b: (0, 0, 0)),
        ],
        out_specs=pl.BlockSpec((1, rows, cout), lambda b: (b, 0, 0)),
        out_shape=jax.ShapeDtypeStruct((B, rows, cout), jnp.float32),
        scratch_shapes=[pltpu.VMEM((1, h + 2, h + 2, cin), jnp.float32)],
    )(x, wt)


def _bn_relu(a, pk):
    return pl.pallas_call(
        _bn_relu_body,
        out_shape=jax.ShapeDtypeStruct(a.shape, jnp.float32),
    )(a, pk)


def _stage_e2_body(h2_ref, w3_ref, w4_ref, p3_ref, p4_ref, mlp_ref, p5_ref,
                   out_ref, s2, s3):
    h2 = h2_ref[...]
    s2[...] = jnp.zeros_like(s2)
    s2[:, 1:31, 1:31, :] = h2
    h3 = _conv_bn_relu(s2[...], w3_ref, p3_ref[...], 2, 15, 256)

    s3[...] = jnp.zeros_like(s3)
    s3[:, 1:16, 1:16, :] = h3
    h4 = _conv_bn_relu(s3[...], w4_ref, p4_ref[...], 2, 8, 256)

    gf = jnp.mean(h4.reshape(B, 64, 256), axis=1)
    lat = lax.dot_general(gf, mlp_ref[...], (((1,), (0,)), ((), ())),
                          preferred_element_type=jnp.float32)
    lat = lat + p5_ref[0:1, :]
    mu = jnp.mean(lat, axis=1, keepdims=True)
    var = jnp.mean((lat - mu) * (lat - mu), axis=1, keepdims=True)
    out_ref[...] = (lat - mu) * lax.rsqrt(var + EPS) * p5_ref[1:2, :] \
        + p5_ref[2:3, :]


def kernel(points, pfn_w, pfn_b, pfn_bn_g, pfn_bn_b,
           cw1, cb1, bg1, bb1, cw2, cb2, bg2, bb2,
           cw3, cb3, bg3, bb3, cw4, cb4, bg4, bb4,
           mlp_w, mlp_b, ln_g, ln_b):
    pts_t = jnp.transpose(points, (0, 2, 1))

    psum, pidx4 = pl.pallas_call(
        _stage_a_body,
        grid=(B, NT),
        in_specs=[pl.BlockSpec((1, 8, PT), lambda b, t: (b, 0, t))],
        out_specs=[
            pl.BlockSpec((1, 4, G64, G64), lambda b, t: (b, 0, 0, 0)),
            pl.BlockSpec((1, 1, 1, PT), lambda b, t: (b, t, 0, 0)),
        ],
        out_shape=[
            jax.ShapeDtypeStruct((B, 4, G64, G64), jnp.float32),
            jax.ShapeDtypeStruct((B, NT, 1, PT), jnp.int32),
        ],
    )(pts_t)

    aug_t, mmat = pl.pallas_call(
        _stage_b_body,
        grid=(B, NT),
        in_specs=[
            pl.BlockSpec((1, 8, PT), lambda b, t: (b, 0, t)),
            pl.BlockSpec((1, 4, G64, G64), lambda b, t: (b, 0, 0, 0)),
        ],
        out_specs=[
            pl.BlockSpec((1, 16, PT), lambda b, t: (b, 0, t)),
            pl.BlockSpec((16, 16), lambda b, t: (0, 0)),
        ],
        out_shape=[
            jax.ShapeDtypeStruct((B, 16, N), jnp.float32),
            jax.ShapeDtypeStruct((16, 16), jnp.float32),
        ],
    )(pts_t, psum)

    wt16 = jnp.zeros((16, 64), jnp.float32).at[0:13, :].set(pfn_w.T)
    pfnp = jnp.zeros((8, 64), jnp.float32)
    pfnp = pfnp.at[0].set(pfn_b).at[1].set(pfn_bn_g).at[2].set(pfn_bn_b)

    feat = pl.pallas_call(
        _stage_c_body,
        grid=(B, NT),
        in_specs=[
            pl.BlockSpec((1, 16, PT), lambda b, t: (b, 0, t)),
            pl.BlockSpec((16, 16), lambda b, t: (0, 0)),
            pl.BlockSpec((16, 64), lambda b, t: (0, 0)),
            pl.BlockSpec((8, 64), lambda b, t: (0, 0)),
        ],
        out_specs=pl.BlockSpec((1, PT, 64), lambda b, t: (b, t, 0)),
        out_shape=jax.ShapeDtypeStruct((B, N, 64), jnp.float32),
    )(aug_t, mmat, wt16, pfnp)

    pfeat1d = _scatter_max(pidx4.reshape(-1), feat.reshape(-1))
    pfeat = pfeat1d.reshape(4, B, 2, TROWS, 64)[:, :, :, :PR, :]
    pfeat = pfeat.reshape(4, B, NPIL, 64)

    w1t = jnp.transpose(cw1, (2, 3, 1, 0)).reshape(9, 64, 64)
    w2t = jnp.transpose(cw2, (2, 3, 1, 0)).reshape(9, 64, 128)
    w3t = jnp.transpose(cw3, (2, 3, 1, 0)).reshape(9, 128, 256)
    w4t = jnp.transpose(cw4, (2, 3, 1, 0)).reshape(9, 256, 256)

    def pack3(a, b_, c):
        p = jnp.zeros((8, a.shape[0]), jnp.float32)
        return p.at[0].set(a).at[1].set(b_).at[2].set(c)

    p1 = pack3(cb1, bg1, bb1)
    p2 = pack3(cb2, bg2, bb2)
    p3 = pack3(cb3, bg3, bb3)
    p4 = pack3(cb4, bg4, bb4)
    p5 = pack3(mlp_b, ln_g, ln_b)

    x0 = pl.pallas_call(
        _stage_merge_body,
        out_shape=jax.ShapeDtypeStruct((B, GH, GW, 64), jnp.float32),
    )(pfeat)

    c1 = _conv_grid(x0, w1t, 1, 60, 64)
    h1 = _bn_relu(c1, p1).reshape(B, 60, 60, 64)
    c2 = _conv_grid(h1, w2t, 2, 30, 128)
    h2 = _bn_relu(c2, p2).reshape(B, 30, 30, 128)

    out = pl.pallas_call(
        _stage_e2_body,
        out_shape=jax.ShapeDtypeStruct((B, 256), jnp.float32),
        scratch_shapes=[
            pltpu.VMEM((B, 32, 32, 128), jnp.float32),
            pltpu.VMEM((B, 18, 18, 256), jnp.float32),
        ],
    )(h2, w3t, w4t, p3, p4, jnp.transpose(mlp_w), p5)

    return out
